# baseline (speedup 1.0000x reference)
import functools

import jax
import jax.numpy as jnp
from jax import lax
from jax.experimental import pallas as pl
from jax.experimental.pallas import tpu as pltpu
from jax.experimental.pallas import tpu_sc as plsc

NC, NS = 2, 16
NW = NC * NS
CHUNK = 128
L = 16


def _mo(x):
  return pl.multiple_of(x, 8)


def kernel(input_ids, token_type_idx, word_emb, pos_emb, type_emb,
           ln_gamma, ln_beta):
  b, s = input_ids.shape
  hidden = word_emb.shape[1]
  n_tok = b * s
  n_chunks = n_tok // CHUNK
  per_w = n_chunks // NW
  tok_w = per_w * CHUNK
  n_grp = CHUNK // L
  hv = hidden // L
  pt_sz = 2 * s * hidden

  mesh = plsc.VectorSubcoreMesh(core_axis_name="c", subcore_axis_name="s",
                                num_cores=NC, num_subcores=NS)

  @functools.partial(
      pl.kernel,
      out_type=jax.ShapeDtypeStruct((n_tok, hidden), jnp.float32),
      mesh=mesh,
      compiler_params=pltpu.CompilerParams(needs_layout_passes=False),
      scratch_types=[
          pltpu.VMEM((tok_w,), jnp.int32),
          pltpu.VMEM((tok_w,), jnp.int32),
          pltpu.VMEM((2, CHUNK, hidden), jnp.float32),
          pltpu.VMEM((pt_sz,), jnp.float32),
          pltpu.VMEM((2 * hidden,), jnp.float32),
          pltpu.VMEM((hidden,), jnp.float32),
          pltpu.VMEM((hidden,), jnp.float32),
          pltpu.VMEM((hidden * L,), jnp.float32),
          pltpu.VMEM((hidden * L,), jnp.float32),
          pltpu.VMEM((hidden * L,), jnp.float32),
          pltpu.SemaphoreType.DMA,
          pltpu.SemaphoreType.DMA,
          pltpu.SemaphoreType.DMA,
          pltpu.SemaphoreType.DMA,
      ],
  )
  def fused(word_hbm, ids_hbm, tt_hbm, pos_hbm, type_hbm, gamma_hbm,
            beta_hbm, out_hbm, ids_v, tt_v, rows_v, pt_v, ty_v, ga_v, be_v,
            gT, bT, tbuf, g0, g1, o0, o1):
    wid = lax.axis_index("s") * NC + lax.axis_index("c")
    gsems = (g0, g1)
    osems = (o0, o1)
    lanes = lax.iota(jnp.int32, L)
    zeros = lanes * 0

    pltpu.sync_copy(ids_hbm.at[pl.ds(_mo(wid * tok_w), tok_w)], ids_v)
    pltpu.sync_copy(tt_hbm.at[pl.ds(_mo(wid * tok_w), tok_w)], tt_v)
    pltpu.sync_copy(pos_hbm, pt_v.at[pl.ds(0, s * hidden)])
    pltpu.sync_copy(pos_hbm, pt_v.at[pl.ds(s * hidden, s * hidden)])
    pltpu.sync_copy(type_hbm, ty_v)
    pltpu.sync_copy(gamma_hbm, ga_v)
    pltpu.sync_copy(beta_hbm, be_v)

    for c0 in (0, 1):
      trow = [ty_v[pl.ds(c0 * hidden + j * L, L)] for j in range(hv)]

      def add_type(si, _, c0=c0, trow=trow):
        base = c0 * s * hidden + si * hidden
        for j in range(hv):
          sl = pl.ds(_mo(base + j * L), L)
          pt_v[sl] = pt_v[sl] + trow[j]
        return 0

      lax.fori_loop(0, s, add_type, 0)

    def bcast_gb(h, _):
      hb = zeros + h
      gT[pl.ds(_mo(h * L), L)] = plsc.load_gather(ga_v, [hb])
      bT[pl.ds(_mo(h * L), L)] = plsc.load_gather(be_v, [hb])
      return 0

    lax.fori_loop(0, hidden, bcast_gb, 0)

    def idx_view(c):
      return ids_v.at[pl.ds(_mo(c * CHUNK), CHUNK)]

    def start_gather(c, bb):
      pltpu.make_async_copy(
          word_hbm.at[idx_view(c)], rows_v.at[bb], gsems[bb]).start()

    def wait_gather(c, bb):
      pltpu.make_async_copy(
          word_hbm.at[idx_view(c)], rows_v.at[bb], gsems[bb]).wait()

    def out_view(c):
      return out_hbm.at[pl.ds(_mo((wid * per_w + c) * CHUNK), CHUNK)]

    def start_out(c, bb):
      pltpu.make_async_copy(rows_v.at[bb], out_view(c), osems[bb]).start()

    def wait_out(c, bb):
      pltpu.make_async_copy(rows_v.at[bb], out_view(c), osems[bb]).wait()

    inv_h = jnp.float32(1.0 / hidden)
    eps = jnp.float32(1e-5)

    def compute_chunk(c, bb):
      rows2 = rows_v.at[bb]
      for g in range(n_grp):
        tok = lanes + g * L
        gtok = wid * tok_w + c * CHUNK + tok
        sv = lax.rem(gtok, s)
        ttv = tt_v[pl.ds(_mo(c * CHUNK + g * L), L)]
        ptbase = ttv * (s * hidden) + sv * hidden

        def pass1(h, carry, tok=tok, ptbase=ptbase, rows2=rows2):
          sum_, sq = carry
          w = plsc.load_gather(rows2, [tok, zeros + h])
          p = plsc.load_gather(pt_v, [ptbase + h])
          val = w + p
          tbuf[pl.ds(_mo(h * L), L)] = val
          return sum_ + val, sq + val * val

        zf = zeros.astype(jnp.float32)
        sum_, sq = lax.fori_loop(0, hidden, pass1, (zf, zf))

        mu = sum_ * inv_h
        var = sq * inv_h - mu * mu + eps
        yi = jnp.int32(0x5F3759DF) - (plsc.bitcast(var, jnp.int32) >> 1)
        y = plsc.bitcast(yi, jnp.float32)
        half = jnp.float32(0.5)
        threehalf = jnp.float32(1.5)
        for _ in range(3):
          y = y * (threehalf - half * var * y * y)

        def pass2(h, _, tok=tok, mu=mu, y=y, rows2=rows2):
          sl = pl.ds(_mo(h * L), L)
          val = tbuf[sl]
          o = (val - mu) * y * gT[sl] + bT[sl]
          plsc.store_scatter(rows2, [tok, zeros + h], o)
          return 0

        lax.fori_loop(0, hidden, pass2, 0)

    start_gather(0, 0)

    def body(k, carry):
      for bb in (0, 1):
        c = 2 * k + bb
        nb = 1 - bb

        @pl.when(c + 1 < per_w)
        def _():
          @pl.when(c >= 1)
          def _():
            wait_out(c - 1, nb)
          start_gather(c + 1, nb)

        wait_gather(c, bb)
        compute_chunk(c, bb)
        start_out(c, bb)
      return carry

    lax.fori_loop(0, per_w // 2, body, 0)
    wait_out(per_w - 2, 0)
    wait_out(per_w - 1, 1)

  ids_flat = input_ids.reshape(n_tok).astype(jnp.int32)
  tt_flat = token_type_idx.reshape(n_tok).astype(jnp.int32)
  out = fused(word_emb, ids_flat, tt_flat, pos_emb[:s].reshape(-1),
              type_emb.reshape(-1), ln_gamma, ln_beta)
  return out.reshape(b, s, hidden)


# speedup vs baseline: 7.9355x; 7.9355x over previous
import functools

import jax
import jax.numpy as jnp
from jax import lax
from jax.experimental import pallas as pl
from jax.experimental.pallas import tpu as pltpu
from jax.experimental.pallas import tpu_sc as plsc

NC, NS = 2, 16
NW = NC * NS
CHUNK = 128


def _sc_gather(word_emb, ids_flat, n_tok):
  n_chunks = n_tok // CHUNK
  per_w = n_chunks // NW
  tok_w = per_w * CHUNK
  hidden = word_emb.shape[1]

  mesh = plsc.VectorSubcoreMesh(core_axis_name="c", subcore_axis_name="s")

  @functools.partial(
      pl.kernel,
      out_type=jax.ShapeDtypeStruct((n_tok, hidden), jnp.float32),
      mesh=mesh,
      scratch_types=[
          pltpu.VMEM((tok_w,), jnp.int32),
          pltpu.VMEM((2, CHUNK, hidden), jnp.float32),
          pltpu.SemaphoreType.DMA,
          pltpu.SemaphoreType.DMA,
          pltpu.SemaphoreType.DMA,
          pltpu.SemaphoreType.DMA,
      ],
  )
  def gather_kernel(word_hbm, ids_hbm, out_hbm, ids_v, rows_v, g0, g1, o0, o1):
    wid = lax.axis_index("s") * NC + lax.axis_index("c")
    base = wid * per_w
    gsems = (g0, g1)
    osems = (o0, o1)

    pltpu.sync_copy(
        ids_hbm.at[pl.ds(pl.multiple_of(wid * tok_w, 8), tok_w)], ids_v)

    def idx_view(c):
      return ids_v.at[pl.ds(pl.multiple_of(c * CHUNK, 8), CHUNK)]

    def start_gather(c, b):
      pltpu.make_async_copy(
          word_hbm.at[idx_view(c)], rows_v.at[b], gsems[b]).start()

    def wait_gather(c, b):
      pltpu.make_async_copy(
          word_hbm.at[idx_view(c)], rows_v.at[b], gsems[b]).wait()

    def out_view(c):
      return out_hbm.at[pl.ds(pl.multiple_of((base + c) * CHUNK, 8), CHUNK)]

    def start_out(c, b):
      pltpu.make_async_copy(rows_v.at[b], out_view(c), osems[b]).start()

    def wait_out(c, b):
      pltpu.make_async_copy(rows_v.at[b], out_view(c), osems[b]).wait()

    start_gather(0, 0)

    def body(k, carry):
      for b in (0, 1):
        c = 2 * k + b
        nb = 1 - b

        @pl.when(c + 1 < per_w)
        def _():
          @pl.when(c >= 1)
          def _():
            wait_out(c - 1, nb)
          start_gather(c + 1, nb)

        wait_gather(c, b)
        start_out(c, b)
      return carry

    lax.fori_loop(0, per_w // 2, body, 0)
    wait_out(per_w - 2, 0)
    wait_out(per_w - 1, 1)

  return gather_kernel(word_emb, ids_flat)


def _tc_ln_kernel(w_ref, tt_ref, pos_ref, type_ref, gamma_ref, beta_ref,
                  out_ref):
  w = w_ref[...]
  tt = tt_ref[...].astype(jnp.float32)[..., None]
  t0 = type_ref[0][None, None, :]
  t1 = type_ref[1][None, None, :]
  x = w + pos_ref[...][None] + (t0 + tt * (t1 - t0))
  mu = jnp.mean(x, axis=-1, keepdims=True)
  xc = x - mu
  var = jnp.mean(xc * xc, axis=-1, keepdims=True)
  y = xc * lax.rsqrt(var + 1e-5)
  out_ref[...] = y * gamma_ref[...][None] + beta_ref[...][None]


def kernel(input_ids, token_type_idx, word_emb, pos_emb, type_emb,
           ln_gamma, ln_beta):
  b, s = input_ids.shape
  hidden = word_emb.shape[1]
  n_tok = b * s

  ids_flat = input_ids.reshape(n_tok).astype(jnp.int32)
  rows = _sc_gather(word_emb, ids_flat, n_tok)
  rows = rows.reshape(b, s, hidden)

  rb = 8
  grid = (b // rb,)
  out = pl.pallas_call(
      _tc_ln_kernel,
      grid=grid,
      in_specs=[
          pl.BlockSpec((rb, s, hidden), lambda i: (i, 0, 0)),
          pl.BlockSpec((rb, s), lambda i: (i, 0)),
          pl.BlockSpec((s, hidden), lambda i: (0, 0)),
          pl.BlockSpec(type_emb.shape, lambda i: (0, 0)),
          pl.BlockSpec((1, hidden), lambda i: (0, 0)),
          pl.BlockSpec((1, hidden), lambda i: (0, 0)),
      ],
      out_specs=pl.BlockSpec((rb, s, hidden), lambda i: (i, 0, 0)),
      out_shape=jax.ShapeDtypeStruct((b, s, hidden), jnp.float32),
  )(rows, token_type_idx.astype(jnp.int32), pos_emb[:s], type_emb,
    ln_gamma.reshape(1, hidden), ln_beta.reshape(1, hidden))
  return out
